# Initial kernel scaffold; baseline (speedup 1.0000x reference)
import jax
import jax.numpy as jnp
from jax.experimental import pallas as pl


def kernel(species, structure_ids, targets, weights):
    raise NotImplementedError("write your pallas kernel here")


# speedup vs baseline: 44.6350x; 44.6350x over previous
import functools

import jax
import jax.numpy as jnp
from jax import lax
from jax.experimental import pallas as pl
from jax.experimental.pallas import tpu as pltpu
from jax.experimental.pallas import tpu_sc as plsc

N_ATOMS = 1_600_000
N_STRUCT = 100_000
ROW = 128
NW = 32
ROWS_PER_W = 392
ROWS = NW * ROWS_PER_W
BLK_ROWS = 56
N_BLKS = ROWS_PER_W // BLK_ROWS
ACC_SLICE = 6_256
N_ACC = 16 * ACC_SLICE
LANES = 16


@functools.partial(
    pl.kernel,
    out_type=jax.ShapeDtypeStruct((2 * N_ACC,), jnp.float32),
    mesh=plsc.VectorSubcoreMesh(core_axis_name="c", subcore_axis_name="s"),
    scratch_types=[
        pltpu.VMEM((BLK_ROWS, ROW), jnp.int32),
        pltpu.VMEM((BLK_ROWS, ROW), jnp.int32),
        pltpu.VMEM((BLK_ROWS, ROW), jnp.float32),
        pltpu.VMEM((128,), jnp.float32),
        pltpu.VMEM((ACC_SLICE,), jnp.float32),
        pltpu.VMEM_SHARED((N_ACC,), jnp.float32),
    ],
    compiler_params=pltpu.CompilerParams(needs_layout_passes=False),
)
def _sc_segment_sum(sp_hbm, sid_hbm, w_hbm, partial_hbm,
                    sp_buf, sid_buf, w_buf, wtab, zbuf, acc_sh):
    c = lax.axis_index("c")
    s = lax.axis_index("s")
    wid = s * 2 + c

    pltpu.sync_copy(w_hbm, wtab)

    zero16 = jnp.zeros((LANES,), jnp.float32)

    def _zero(i, carry):
        zbuf[pl.ds(i * LANES, LANES)] = zero16
        return carry

    lax.fori_loop(0, ACC_SLICE // LANES, _zero, 0)
    pltpu.sync_copy(zbuf, acc_sh.at[pl.ds(s * ACC_SLICE, ACC_SLICE)])
    plsc.subcore_barrier()

    row0 = wid * ROWS_PER_W

    def _block(b, carry):
        r0 = row0 + b * BLK_ROWS
        pltpu.sync_copy(sp_hbm.at[pl.ds(r0, BLK_ROWS)], sp_buf)
        pltpu.sync_copy(sid_hbm.at[pl.ds(r0, BLK_ROWS)], sid_buf)

        def _row(r, inner):
            for g in range(ROW // LANES):
                spv = sp_buf[r, pl.ds(g * LANES, LANES)]
                w_buf[r, pl.ds(g * LANES, LANES)] = plsc.load_gather(
                    wtab, [spv])
            pltpu.sync_copy(w_buf.at[r], acc_sh.at[sid_buf.at[r]], add=True)
            return inner

        lax.fori_loop(0, BLK_ROWS, _row, 0)
        return carry

    lax.fori_loop(0, N_BLKS, _block, 0)
    plsc.subcore_barrier()

    pltpu.sync_copy(acc_sh.at[pl.ds(s * ACC_SLICE, ACC_SLICE)], zbuf)
    pltpu.sync_copy(zbuf,
                    partial_hbm.at[pl.ds(c * N_ACC + s * ACC_SLICE,
                                         ACC_SLICE)])


def _combine_tc(t_ref, p_ref, o_ref):
    o_ref[...] = t_ref[...] - p_ref[0] - p_ref[1]


def kernel(species, structure_ids, targets, weights):
    pad = ROWS * ROW - N_ATOMS
    sp2 = jnp.concatenate(
        [species.astype(jnp.int32), jnp.zeros((pad,), jnp.int32)]
    ).reshape(ROWS, ROW)
    sid2 = jnp.concatenate(
        [structure_ids.astype(jnp.int32),
         jnp.full((pad,), N_STRUCT, jnp.int32)]
    ).reshape(ROWS, ROW)
    wtab = jnp.zeros((128,), jnp.float32).at[: weights.shape[0]].set(
        weights[:, 0])
    partial = _sc_segment_sum(sp2, sid2, wtab)

    t_pad = jnp.zeros((N_ACC,), jnp.float32).at[:N_STRUCT].set(
        targets[:, 0]).reshape(N_ACC // 128, 128)
    p3 = partial.reshape(2, N_ACC // 128, 128)
    out = pl.pallas_call(
        _combine_tc,
        out_shape=jax.ShapeDtypeStruct((N_ACC // 128, 128), jnp.float32),
    )(t_pad, p3)
    return out.reshape(-1)[:N_STRUCT].reshape(N_STRUCT, 1)
